# baseline (speedup 1.0000x reference)
import functools
import jax
import jax.numpy as jnp
from jax.experimental import pallas as pl
from jax.experimental.pallas import tpu as pltpu

B = 4
N = 512
D = 3
HID = 32
NUM_ATOM_TYPE = 54
NUM_AA_TYPE = 20
NUM_AA_MAX = 32
NUM_VALID_SEQ = 17
NL = 2
CR = 15.0 / NL
CP = 8
TI = 128
NT = N // TI
NN = B * N
PK = 4
LW = PK * HID
NQ = N // PK
R = TI * NQ
F32 = jnp.float32
BF = jnp.bfloat16


def _b(v):
    return v.astype(BF)


def _silu(v):
    return v * jax.nn.sigmoid(v)


def _embed_body(at_ref, ap_ref, aty_ref, sl_ref, tt_ref,
                wat_ref, wap_ref, waty_ref, wsl_ref, wt_ref, eb_ref, h_ref):
    def onehot(idx, k):
        iota = jax.lax.broadcasted_iota(jnp.int32, (NN, k), 1)
        return jnp.where(iota == idx, 1.0, 0.0).astype(F32)

    at = jnp.maximum(at_ref[...] - 1, 0)
    ap = jnp.maximum(ap_ref[...] - 1, 0)
    aty = jnp.maximum(aty_ref[...] - 1, 0)
    sl = jnp.maximum(sl_ref[...] - (NUM_AA_MAX - NUM_VALID_SEQ + 1), 0)
    h = jnp.dot(_b(onehot(at, NUM_ATOM_TYPE)), _b(wat_ref[...]), preferred_element_type=F32)
    h = h + jnp.dot(_b(onehot(ap, NUM_AA_MAX)), _b(wap_ref[...]), preferred_element_type=F32)
    h = h + jnp.dot(_b(onehot(aty, NUM_AA_TYPE)), _b(waty_ref[...]), preferred_element_type=F32)
    h = h + jnp.dot(_b(onehot(sl, NUM_VALID_SEQ)), _b(wsl_ref[...]), preferred_element_type=F32)
    h = h + _b(tt_ref[...]).astype(F32) * _b(wt_ref[...]).astype(F32)
    h_ref[...] = h + eb_ref[...]


def _layer_impl(first_layer, hi_ref, hpk_ref, ci_ref, cpk_ref, xi_ref, xpk_ref,
                eW1a_ref, eW1b_ref, eW1r_ref, eW1e_ref, eB1_ref,
                eW2_ref, eB2_ref, attW_ref, attB_ref,
                cW1_ref, cB1_ref, cW2_ref,
                nW1h_ref, nW1n_ref, nB1_ref, nW2_ref, nB2_ref,
                h_out_ref, c_out_ref):
    ti = pl.program_id(1)
    i0 = ti * TI

    hi = hi_ref[...]
    hpk = hpk_ref[...]
    ci = ci_ref[...]
    cpk = cpk_ref[...]
    cds = [ci[:, d:d + 1][:, :, None] - cpk[d][None] for d in range(D)]
    radial = cds[0] * cds[0] + cds[1] * cds[1] + cds[2] * cds[2]
    if not first_layer:
        xi = xi_ref[...]
        xpk = xpk_ref[...]
        xds = [xi[:, d:d + 1][:, :, None] - xpk[d][None] for d in range(D)]
        ea = xds[0] * xds[0] + xds[1] * xds[1] + xds[2] * xds[2]
    invn = 1.0 / (jnp.sqrt(radial + 1e-8) + 1.0)

    hr = jnp.dot(_b(hi), _b(eW1a_ref[...]), preferred_element_type=F32)
    hr_t = jnp.tile(hr, (1, PK))[:, None, :]
    hc = jnp.dot(_b(hpk), _b(eW1b_ref[...]), preferred_element_type=F32)
    if first_layer:
        pre = (hr_t + hc[None]
               + _b(radial).astype(F32)
               * (_b(eW1r_ref[...]).astype(F32) + _b(eW1e_ref[...]).astype(F32))
               + eB1_ref[...])
    else:
        pre = (hr_t + hc[None]
               + _b(radial).astype(F32) * _b(eW1r_ref[...]).astype(F32)
               + _b(ea).astype(F32) * _b(eW1e_ref[...]).astype(F32) + eB1_ref[...])
    ef1 = _silu(pre).reshape(R, LW)
    ef2 = _silu(jnp.dot(_b(ef1), _b(eW2_ref[...]), preferred_element_type=F32)
                + eB2_ref[...])

    a = jnp.dot(_b(ef2), _b(attW_ref[...]), preferred_element_type=F32) + attB_ref[...]
    r3 = jax.lax.broadcasted_iota(jnp.int32, (TI, NQ, LW), 0) + i0
    q3 = jax.lax.broadcasted_iota(jnp.int32, (TI, NQ, LW), 1)
    l3 = jax.lax.broadcasted_iota(jnp.int32, (TI, NQ, LW), 2)
    mask = jnp.where(q3 * PK + l3 // HID != r3, 1.0, 0.0).astype(F32)
    g = jax.nn.sigmoid(a).reshape(TI, NQ, LW) * mask
    ef = ef2.reshape(TI, NQ, LW) * g

    cc = jnp.dot(_b(ef.reshape(R, LW)), _b(cW1_ref[...]), preferred_element_type=F32)
    m1 = _silu(cc + cB1_ref[...])
    s = jnp.tanh(jnp.dot(_b(m1), _b(cW2_ref[...]), preferred_element_type=F32))
    w = invn * s.reshape(TI, NQ, LW)

    def fold(v):
        return (v[:, 0 * HID:1 * HID] + v[:, 1 * HID:2 * HID]
                + v[:, 2 * HID:3 * HID] + v[:, 3 * HID:4 * HID])

    delta_cols = []
    for d in range(D):
        t = fold(jnp.sum(cds[d] * w, axis=1))
        delta_cols.append(t[:, 0:1] * CR)
    delta = jnp.concatenate(delta_cols + [jnp.zeros((TI, CP - D), F32)], axis=1)
    c_out_ref[...] = ci + delta

    nagg = fold(jnp.sum(ef, axis=1))
    mid = _silu(jnp.dot(_b(hi), _b(nW1h_ref[...]), preferred_element_type=F32)
                + jnp.dot(_b(nagg), _b(nW1n_ref[...]), preferred_element_type=F32)
                + nB1_ref[...])
    h_out_ref[...] = hi + jnp.dot(_b(mid), _b(nW2_ref[...]), preferred_element_type=F32) + nB2_ref[...]


def _final_body(c_ref, x_ref, v_ref):
    vel = c_ref[...] - x_ref[...]
    vel3 = vel.reshape(B, N, CP)
    mean = jnp.sum(vel3, axis=1, keepdims=True) * (1.0 / N)
    v_ref[...] = (vel3 - mean).reshape(NN, CP)


def _full(spec_shape):
    return pl.BlockSpec(spec_shape, lambda *_: tuple(0 for _ in spec_shape))


def _pack_coords(c):
    ct = c[:, :D].T.reshape(D, NN // PK, PK, 1)
    return jnp.broadcast_to(ct, (D, NN // PK, PK, HID)).reshape(D, NN // PK, LW)


def kernel(t, x, atom_type, aa_pos, aa_type, seq_len, emb_W, emb_b,
           eW1, eB1, eW2, eB2, attW, attB, nW1, nB1, nW2, nB2, cW1, cB1, cW2):
    xf = x.reshape(NN, D)
    xp = jnp.concatenate([xf, jnp.zeros((NN, CP - D), F32)], axis=1)
    xpk = _pack_coords(xp)

    at = atom_type.astype(jnp.int32).reshape(NN, 1)
    ap = aa_pos.astype(jnp.int32).reshape(NN, 1)
    aty = aa_type.astype(jnp.int32).reshape(NN, 1)
    slb = jnp.broadcast_to(seq_len.astype(jnp.int32), (B, N)).reshape(NN, 1)
    tt = jnp.broadcast_to(t, (B, N)).reshape(NN, 1)

    o = NUM_ATOM_TYPE
    wat = emb_W[:o]
    wap = emb_W[o:o + NUM_AA_MAX]; o += NUM_AA_MAX
    waty = emb_W[o:o + NUM_AA_TYPE]; o += NUM_AA_TYPE
    wsl = emb_W[o:o + NUM_VALID_SEQ]; o += NUM_VALID_SEQ
    wt = emb_W[o:o + 1]

    h = pl.pallas_call(
        _embed_body,
        grid=(1,),
        in_specs=[_full((NN, 1))] * 5 + [
            _full((NUM_ATOM_TYPE, HID)), _full((NUM_AA_MAX, HID)),
            _full((NUM_AA_TYPE, HID)), _full((NUM_VALID_SEQ, HID)),
            _full((1, HID)), _full((1, HID)),
        ],
        out_specs=_full((NN, HID)),
        out_shape=jax.ShapeDtypeStruct((NN, HID), F32),
    )(at, ap, aty, slb, tt, wat, wap, waty, wsl, wt, emb_b.reshape(1, HID))

    c = xp
    row_spec_h = pl.BlockSpec((TI, HID), lambda b, ti: (b * NT + ti, 0))
    row_spec_c = pl.BlockSpec((TI, CP), lambda b, ti: (b * NT + ti, 0))
    pk_spec_h = pl.BlockSpec((NQ, LW), lambda b, ti: (b, 0))
    pk_spec_c = pl.BlockSpec((D, NQ, LW), lambda b, ti: (0, b, 0))

    def make_layer_call(first_layer):
        return pl.pallas_call(
        functools.partial(_layer_impl, first_layer),
        grid=(B, NT),
        in_specs=[
            row_spec_h, pk_spec_h, row_spec_c, pk_spec_c, row_spec_c, pk_spec_c,
            _full((HID, HID)), _full((LW, LW)), _full((1, LW)),
            _full((1, LW)), _full((1, LW)),
            _full((LW, LW)), _full((1, LW)), _full((LW, LW)), _full((1, 1)),
            _full((LW, LW)), _full((1, LW)), _full((LW, LW)),
            _full((HID, HID)), _full((HID, HID)), _full((1, HID)),
            _full((HID, HID)), _full((1, HID)),
        ],
        out_specs=[row_spec_h, row_spec_c],
        out_shape=[jax.ShapeDtypeStruct((NN, HID), F32),
                   jax.ShapeDtypeStruct((NN, CP), F32)],
        )

    layer_calls = [make_layer_call(l == 0) for l in range(NL)]
    eye4 = jnp.eye(PK, dtype=F32)
    ones_row = jnp.ones((1, HID), F32)

    def tile4(v):
        return jnp.tile(v.reshape(1, HID), (1, PK))

    for l in range(NL):
        hpk = h.reshape(NN // PK, LW)
        cpk = _pack_coords(c)
        h, c = layer_calls[l](
            h, hpk, c, cpk, xp, xpk,
            eW1[l, :HID],
            jnp.kron(eye4, eW1[l, HID:2 * HID]),
            tile4(eW1[l, 2 * HID]), tile4(eW1[l, 2 * HID + 1]),
            tile4(eB1[l]),
            jnp.kron(eye4, eW2[l]), tile4(eB2[l]),
            jnp.kron(eye4, attW[l] @ ones_row), attB[l].reshape(1, 1),
            jnp.kron(eye4, cW1[l]), tile4(cB1[l]),
            jnp.kron(eye4, cW2[l] @ ones_row),
            nW1[l, :HID], nW1[l, HID:], nB1[l].reshape(1, HID),
            nW2[l], nB2[l].reshape(1, HID),
        )

    velp = pl.pallas_call(
        _final_body,
        grid=(1,),
        in_specs=[_full((NN, CP)), _full((NN, CP))],
        out_specs=_full((NN, CP)),
        out_shape=jax.ShapeDtypeStruct((NN, CP), F32),
    )(c, xp)

    return velp[:, :D].reshape(B, N, D).reshape(B, N * D)


# speedup vs baseline: 1.0370x; 1.0370x over previous
import functools
import jax
import jax.numpy as jnp
from jax.experimental import pallas as pl

B = 4
N = 512
D = 3
HID = 32
NUM_ATOM_TYPE = 54
NUM_AA_TYPE = 20
NUM_AA_MAX = 32
NUM_VALID_SEQ = 17
NL = 2
CR = 15.0 / NL
CP = 8
TI = 64
NT = N // TI
NN = B * N
PK = 4
LW = PK * HID
NQ = N // PK
R = TI * NQ
F32 = jnp.float32
BF = jnp.bfloat16


def _b(v):
    return v.astype(BF)


def _silu(v):
    return v * jax.nn.sigmoid(v)


def _embed_body(at_ref, ap_ref, aty_ref, sl_ref, tt_ref,
                wat_ref, wap_ref, waty_ref, wsl_ref, wt_ref, eb_ref, h_ref):
    def onehot(idx, k):
        iota = jax.lax.broadcasted_iota(jnp.int32, (NN, k), 1)
        return jnp.where(iota == idx, 1.0, 0.0).astype(F32)

    at = jnp.maximum(at_ref[...] - 1, 0)
    ap = jnp.maximum(ap_ref[...] - 1, 0)
    aty = jnp.maximum(aty_ref[...] - 1, 0)
    sl = jnp.maximum(sl_ref[...] - (NUM_AA_MAX - NUM_VALID_SEQ + 1), 0)
    h = jnp.dot(_b(onehot(at, NUM_ATOM_TYPE)), _b(wat_ref[...]), preferred_element_type=F32)
    h = h + jnp.dot(_b(onehot(ap, NUM_AA_MAX)), _b(wap_ref[...]), preferred_element_type=F32)
    h = h + jnp.dot(_b(onehot(aty, NUM_AA_TYPE)), _b(waty_ref[...]), preferred_element_type=F32)
    h = h + jnp.dot(_b(onehot(sl, NUM_VALID_SEQ)), _b(wsl_ref[...]), preferred_element_type=F32)
    h = h + _b(tt_ref[...]).astype(F32) * _b(wt_ref[...]).astype(F32)
    h_ref[...] = h + eb_ref[...]


def _layer_impl(first_layer, hi_ref, hpk_ref, ci_ref, cpk_ref, xi_ref, xpk_ref,
                eW1a_ref, eW1b_ref, eW1r_ref, eW1e_ref, eB1_ref,
                eW2_ref, eB2_ref, attW_ref, attB_ref,
                cW1_ref, cB1_ref, cW2_ref,
                nW1h_ref, nW1n_ref, nB1_ref, nW2_ref, nB2_ref,
                h_out_ref, c_out_ref):
    ti = pl.program_id(1)
    i0 = ti * TI

    hi = hi_ref[...]
    hpk = hpk_ref[...]
    ci = ci_ref[...]
    cpk = cpk_ref[...]
    cds = [ci[:, d:d + 1][:, :, None] - cpk[d][None] for d in range(D)]
    radial = cds[0] * cds[0] + cds[1] * cds[1] + cds[2] * cds[2]
    if not first_layer:
        xi = xi_ref[...]
        xpk = xpk_ref[...]
        xds = [xi[:, d:d + 1][:, :, None] - xpk[d][None] for d in range(D)]
        ea = xds[0] * xds[0] + xds[1] * xds[1] + xds[2] * xds[2]
    invn = 1.0 / (jnp.sqrt(radial + 1e-8) + 1.0)

    hr = jnp.dot(_b(hi), _b(eW1a_ref[...]), preferred_element_type=F32)
    hr_t = jnp.tile(hr, (1, PK))[:, None, :]
    hc = jnp.dot(_b(hpk), _b(eW1b_ref[...]), preferred_element_type=F32)
    if first_layer:
        pre = (hr_t + hc[None]
               + _b(radial).astype(F32)
               * (_b(eW1r_ref[...]).astype(F32) + _b(eW1e_ref[...]).astype(F32))
               + eB1_ref[...])
    else:
        pre = (hr_t + hc[None]
               + _b(radial).astype(F32) * _b(eW1r_ref[...]).astype(F32)
               + _b(ea).astype(F32) * _b(eW1e_ref[...]).astype(F32) + eB1_ref[...])
    ef1 = _silu(pre).reshape(R, LW)
    ef2 = _silu(jnp.dot(_b(ef1), _b(eW2_ref[...]), preferred_element_type=F32)
                + eB2_ref[...])

    a = jnp.dot(_b(ef2), _b(attW_ref[...]), preferred_element_type=F32) + attB_ref[...]
    r3 = jax.lax.broadcasted_iota(jnp.int32, (TI, NQ, LW), 0) + i0
    q3 = jax.lax.broadcasted_iota(jnp.int32, (TI, NQ, LW), 1)
    l3 = jax.lax.broadcasted_iota(jnp.int32, (TI, NQ, LW), 2)
    mask = jnp.where(q3 * PK + l3 // HID != r3, 1.0, 0.0).astype(F32)
    g = jax.nn.sigmoid(a).reshape(TI, NQ, LW) * mask
    ef = ef2.reshape(TI, NQ, LW) * g

    cc = jnp.dot(_b(ef.reshape(R, LW)), _b(cW1_ref[...]), preferred_element_type=F32)
    m1 = _silu(cc + cB1_ref[...])
    s = jnp.tanh(jnp.dot(_b(m1), _b(cW2_ref[...]), preferred_element_type=F32))
    w = invn * s.reshape(TI, NQ, LW)

    def fold(v):
        return (v[:, 0 * HID:1 * HID] + v[:, 1 * HID:2 * HID]
                + v[:, 2 * HID:3 * HID] + v[:, 3 * HID:4 * HID])

    delta_cols = []
    for d in range(D):
        t = fold(jnp.sum(cds[d] * w, axis=1))
        delta_cols.append(t[:, 0:1] * CR)
    delta = jnp.concatenate(delta_cols + [jnp.zeros((TI, CP - D), F32)], axis=1)
    c_out_ref[...] = ci + delta

    nagg = fold(jnp.sum(ef, axis=1))
    mid = _silu(jnp.dot(_b(hi), _b(nW1h_ref[...]), preferred_element_type=F32)
                + jnp.dot(_b(nagg), _b(nW1n_ref[...]), preferred_element_type=F32)
                + nB1_ref[...])
    h_out_ref[...] = hi + jnp.dot(_b(mid), _b(nW2_ref[...]), preferred_element_type=F32) + nB2_ref[...]


def _final_body(c_ref, x_ref, v_ref):
    vel = c_ref[...] - x_ref[...]
    vel3 = vel.reshape(B, N, CP)
    mean = jnp.sum(vel3, axis=1, keepdims=True) * (1.0 / N)
    v_ref[...] = (vel3 - mean).reshape(NN, CP)


def _full(spec_shape):
    return pl.BlockSpec(spec_shape, lambda *_: tuple(0 for _ in spec_shape))


def _pack_coords(c):
    ct = c[:, :D].T.reshape(D, NN // PK, PK, 1)
    return jnp.broadcast_to(ct, (D, NN // PK, PK, HID)).reshape(D, NN // PK, LW)


def kernel(t, x, atom_type, aa_pos, aa_type, seq_len, emb_W, emb_b,
           eW1, eB1, eW2, eB2, attW, attB, nW1, nB1, nW2, nB2, cW1, cB1, cW2):
    xf = x.reshape(NN, D)
    xp = jnp.concatenate([xf, jnp.zeros((NN, CP - D), F32)], axis=1)
    xpk = _pack_coords(xp)

    at = atom_type.astype(jnp.int32).reshape(NN, 1)
    ap = aa_pos.astype(jnp.int32).reshape(NN, 1)
    aty = aa_type.astype(jnp.int32).reshape(NN, 1)
    slb = jnp.broadcast_to(seq_len.astype(jnp.int32), (B, N)).reshape(NN, 1)
    tt = jnp.broadcast_to(t, (B, N)).reshape(NN, 1)

    o = NUM_ATOM_TYPE
    wat = emb_W[:o]
    wap = emb_W[o:o + NUM_AA_MAX]; o += NUM_AA_MAX
    waty = emb_W[o:o + NUM_AA_TYPE]; o += NUM_AA_TYPE
    wsl = emb_W[o:o + NUM_VALID_SEQ]; o += NUM_VALID_SEQ
    wt = emb_W[o:o + 1]

    h = pl.pallas_call(
        _embed_body,
        grid=(1,),
        in_specs=[_full((NN, 1))] * 5 + [
            _full((NUM_ATOM_TYPE, HID)), _full((NUM_AA_MAX, HID)),
            _full((NUM_AA_TYPE, HID)), _full((NUM_VALID_SEQ, HID)),
            _full((1, HID)), _full((1, HID)),
        ],
        out_specs=_full((NN, HID)),
        out_shape=jax.ShapeDtypeStruct((NN, HID), F32),
    )(at, ap, aty, slb, tt, wat, wap, waty, wsl, wt, emb_b.reshape(1, HID))

    c = xp
    row_spec_h = pl.BlockSpec((TI, HID), lambda b, ti: (b * NT + ti, 0))
    row_spec_c = pl.BlockSpec((TI, CP), lambda b, ti: (b * NT + ti, 0))
    pk_spec_h = pl.BlockSpec((NQ, LW), lambda b, ti: (b, 0))
    pk_spec_c = pl.BlockSpec((D, NQ, LW), lambda b, ti: (0, b, 0))

    def make_layer_call(first_layer):
        return pl.pallas_call(
        functools.partial(_layer_impl, first_layer),
        grid=(B, NT),
        in_specs=[
            row_spec_h, pk_spec_h, row_spec_c, pk_spec_c, row_spec_c, pk_spec_c,
            _full((HID, HID)), _full((LW, LW)), _full((1, LW)),
            _full((1, LW)), _full((1, LW)),
            _full((LW, LW)), _full((1, LW)), _full((LW, LW)), _full((1, 1)),
            _full((LW, LW)), _full((1, LW)), _full((LW, LW)),
            _full((HID, HID)), _full((HID, HID)), _full((1, HID)),
            _full((HID, HID)), _full((1, HID)),
        ],
        out_specs=[row_spec_h, row_spec_c],
        out_shape=[jax.ShapeDtypeStruct((NN, HID), F32),
                   jax.ShapeDtypeStruct((NN, CP), F32)],
        )

    layer_calls = [make_layer_call(l == 0) for l in range(NL)]
    eye4 = jnp.eye(PK, dtype=F32)
    ones_row = jnp.ones((1, HID), F32)

    def tile4(v):
        return jnp.tile(v.reshape(1, HID), (1, PK))

    for l in range(NL):
        hpk = h.reshape(NN // PK, LW)
        cpk = _pack_coords(c)
        h, c = layer_calls[l](
            h, hpk, c, cpk, xp, xpk,
            eW1[l, :HID],
            jnp.kron(eye4, eW1[l, HID:2 * HID]),
            tile4(eW1[l, 2 * HID]), tile4(eW1[l, 2 * HID + 1]),
            tile4(eB1[l]),
            jnp.kron(eye4, eW2[l]), tile4(eB2[l]),
            jnp.kron(eye4, attW[l] @ ones_row), attB[l].reshape(1, 1),
            jnp.kron(eye4, cW1[l]), tile4(cB1[l]),
            jnp.kron(eye4, cW2[l] @ ones_row),
            nW1[l, :HID], nW1[l, HID:], nB1[l].reshape(1, HID),
            nW2[l], nB2[l].reshape(1, HID),
        )

    velp = pl.pallas_call(
        _final_body,
        grid=(1,),
        in_specs=[_full((NN, CP)), _full((NN, CP))],
        out_specs=_full((NN, CP)),
        out_shape=jax.ShapeDtypeStruct((NN, CP), F32),
    )(c, xp)

    return velp[:, :D].reshape(B, N, D).reshape(B, N * D)
